# baseline (speedup 1.0000x reference)
import jax
import jax.numpy as jnp
from jax import lax
from jax.experimental import pallas as pl
from jax.experimental.pallas import tpu as pltpu
from jax.experimental.pallas import tpu_sc as plsc

N = 10000
E = 160000
NS = 16
NC = 2
RB = 1000
F = 128

NJ = N + 16
RPT = 624
TAIL = N - NS * RPT
TAIL_OFF = NS * RPT

EPT = E // NS
W_PROP = 96
WPB = 18
NBLK = 6
EPT_PAD = W_PROP * WPB * NBLK
NW_PROP = WPB * NBLK

W_DEG = 128
EPW = E // (NC * NS)
EPW_PAD = 5120
NW_DEG = EPW_PAD // W_DEG

_mesh = plsc.VectorSubcoreMesh(core_axis_name="c", subcore_axis_name="s")


def _copy_rows(sid, mk_src, mk_dst):
    pltpu.sync_copy(mk_src(sid * RPT, RPT), mk_dst(sid * RPT, RPT))

    @pl.when(sid == 0)
    def _():
        pltpu.sync_copy(mk_src(TAIL_OFF, TAIL), mk_dst(TAIL_OFF, TAIL))


def _deg_body(dst_hbm, ones_hbm, zeros_hbm, out_hbm, dstv, onesv, deg_s, sem):
    cid = lax.axis_index("c")
    sid = lax.axis_index("s")
    wid = cid * NS + sid
    pltpu.sync_copy(ones_hbm, onesv)
    pltpu.sync_copy(dst_hbm.at[wid], dstv)
    _copy_rows(sid, lambda o, n: zeros_hbm.at[pl.ds(o, n)],
               lambda o, n: deg_s.at[pl.ds(o, n)])
    plsc.subcore_barrier()

    descs = []
    for w in range(NW_DEG):
        descs.append(pltpu.async_copy(onesv, deg_s.at[dstv.at[w]], sem,
                                      add=True))
    for d in descs:
        d.wait()
    plsc.subcore_barrier()
    _copy_rows(sid, lambda o, n: deg_s.at[pl.ds(o, n)],
               lambda o, n: out_hbm.at[cid, pl.ds(o, n)])


_deg_call = pl.kernel(
    _deg_body,
    out_type=jax.ShapeDtypeStruct((NC, N, F), jnp.float32),
    mesh=_mesh,
    scratch_types=[
        pltpu.VMEM((NW_DEG, W_DEG), jnp.int32),
        pltpu.VMEM((W_DEG, F), jnp.float32),
        pltpu.VMEM_SHARED((NJ, F), jnp.float32),
        pltpu.SemaphoreType.DMA,
    ],
)


def _make_prop(C):
    CPC = C // NC

    def body(h_hbm, srcs_hbm, dst_hbm, out_hbm,
             s0s, s0d, s1s, s1d, r0, r1, r2, accum_s,
             gs0, gs1, gs2, ss0, ss1, ss2, is0, is1):
        cid = lax.axis_index("c")
        sid = lax.axis_index("s")
        bufs = ((r0, gs0, ss0), (r1, gs1, ss1), (r2, gs2, ss2))

        for k in range(CPC):
            chunk = cid * CPC + k

            def ld_blk(b, ssrc, sdst, sem):
                return (pltpu.make_async_copy(srcs_hbm.at[chunk, sid, b],
                                              ssrc, sem),
                        pltpu.make_async_copy(dst_hbm.at[sid, b], sdst, sem))

            def triple(ssrc, sdst, r, nxt):
                for j, (buf, gsem, ssem) in enumerate(bufs):
                    pltpu.make_async_copy(h_hbm.at[ssrc.at[r + j]], buf,
                                          gsem).wait()
                    pltpu.async_copy(buf, accum_s.at[sdst.at[r + j]], ssem,
                                     add=True)
                for j, (buf, gsem, ssem) in enumerate(bufs):
                    pltpu.make_async_copy(buf, accum_s.at[sdst.at[r + j]],
                                          ssem).wait()
                    if nxt is not None:
                        nsrc, nr = nxt
                        pltpu.async_copy(h_hbm.at[nsrc.at[nr + j]], buf,
                                         gsem)

            def block(ssrc, sdst, nxt_wait, cross):
                for t in range(WPB // 3 - 1):
                    triple(ssrc, sdst, 3 * t, (ssrc, 3 * t + 3))
                if nxt_wait is not None:
                    for d in nxt_wait:
                        d.wait()
                triple(ssrc, sdst, WPB - 3, cross)

            _copy_rows(sid, lambda o, n: h_hbm.at[pl.ds(chunk * N + o, n)],
                       lambda o, n: accum_s.at[pl.ds(o, n)])
            plsc.subcore_barrier()

            for d in ld_blk(0, s0s, s0d, is0):
                d.start()
            for d in ld_blk(0, s0s, s0d, is0):
                d.wait()
            for d in ld_blk(1, s1s, s1d, is1):
                d.start()
            for j, (buf, gsem, _) in enumerate(bufs):
                pltpu.async_copy(h_hbm.at[s0s.at[j]], buf, gsem)

            block(s0s, s0d, ld_blk(1, s1s, s1d, is1), (s1s, 0))

            def fbody(i, carry):
                b1 = 2 * i + 1
                for d in ld_blk(b1 + 1, s0s, s0d, is0):
                    d.start()
                block(s1s, s1d, ld_blk(b1 + 1, s0s, s0d, is0), (s0s, 0))
                b2 = 2 * i + 2
                for d in ld_blk(b2 + 1, s1s, s1d, is1):
                    d.start()
                block(s0s, s0d, ld_blk(b2 + 1, s1s, s1d, is1), (s1s, 0))
                return carry

            lax.fori_loop(0, (NBLK - 2) // 2, fbody, 0)

            block(s1s, s1d, None, None)

            plsc.subcore_barrier()
            _copy_rows(sid, lambda o, n: accum_s.at[pl.ds(o, n)],
                       lambda o, n: out_hbm.at[chunk, pl.ds(o, n)])
            plsc.subcore_barrier()

    return pl.kernel(
        body,
        out_type=jax.ShapeDtypeStruct((C, N, F), jnp.float32),
        mesh=_mesh,
        scratch_types=[
            pltpu.VMEM((WPB, W_PROP), jnp.int32),
            pltpu.VMEM((WPB, W_PROP), jnp.int32),
            pltpu.VMEM((WPB, W_PROP), jnp.int32),
            pltpu.VMEM((WPB, W_PROP), jnp.int32),
            pltpu.VMEM((W_PROP, F), jnp.float32),
            pltpu.VMEM((W_PROP, F), jnp.float32),
            pltpu.VMEM((W_PROP, F), jnp.float32),
            pltpu.VMEM_SHARED((NJ, F), jnp.float32),
            pltpu.SemaphoreType.DMA,
            pltpu.SemaphoreType.DMA,
            pltpu.SemaphoreType.DMA,
            pltpu.SemaphoreType.DMA,
            pltpu.SemaphoreType.DMA,
            pltpu.SemaphoreType.DMA,
            pltpu.SemaphoreType.DMA,
            pltpu.SemaphoreType.DMA,
        ],
    )


_prop4 = _make_prop(4)
_prop2 = _make_prop(2)


def _mm1_body(x_ref, w_ref, out_ref):
    out_ref[0] = jnp.dot(x_ref[...], w_ref[0],
                         preferred_element_type=jnp.float32)


def _mm1(x, w_chunks):
    cout = w_chunks.shape[0]
    return pl.pallas_call(
        _mm1_body,
        grid=(N // RB, cout),
        in_specs=[
            pl.BlockSpec((RB, 256), lambda r, c: (r, 0)),
            pl.BlockSpec((1, 256, F), lambda r, c: (c, 0, 0)),
        ],
        out_specs=pl.BlockSpec((1, RB, F), lambda r, c: (c, r, 0)),
        out_shape=jax.ShapeDtypeStruct((cout, N, F), jnp.float32),
    )(x, w_chunks)


def _scale1_body(h_ref, degp_ref, hout_ref, dinv_ref):
    d = 1.0 + degp_ref[0, :, 0:1] + degp_ref[1, :, 0:1]
    dinv = lax.rsqrt(d)
    hout_ref[0] = h_ref[0] * dinv
    dinv_ref[...] = dinv


def _scale1(h, degp):
    cout = h.shape[0]
    return pl.pallas_call(
        _scale1_body,
        grid=(N // RB, cout),
        in_specs=[
            pl.BlockSpec((1, RB, F), lambda r, c: (c, r, 0)),
            pl.BlockSpec((NC, RB, F), lambda r, c: (0, r, 0)),
        ],
        out_specs=[
            pl.BlockSpec((1, RB, F), lambda r, c: (c, r, 0)),
            pl.BlockSpec((RB, 1), lambda r, c: (r, 0)),
        ],
        out_shape=[
            jax.ShapeDtypeStruct((cout, N, F), jnp.float32),
            jax.ShapeDtypeStruct((N, 1), jnp.float32),
        ],
    )(h, degp)


def _make_mm23_body(cin):
    def body(xc_ref, dinv_ref, b_ref, w_ref, out_ref):
        dinv = dinv_ref[...]
        acc = jnp.zeros((RB, F), jnp.float32)
        for k in range(cin):
            xk = jnp.maximum(xc_ref[k] * dinv + b_ref[k], 0.0)
            acc = acc + jnp.dot(xk, w_ref[0, k],
                                preferred_element_type=jnp.float32)
        out_ref[0] = acc * dinv
    return body


def _mm23(xc, dinv, b_in, w_chunks):
    cout, cin = w_chunks.shape[0], w_chunks.shape[1]
    return pl.pallas_call(
        _make_mm23_body(cin),
        grid=(N // RB, cout),
        in_specs=[
            pl.BlockSpec((cin, RB, F), lambda r, c: (0, r, 0)),
            pl.BlockSpec((RB, 1), lambda r, c: (r, 0)),
            pl.BlockSpec((cin, 1, F), lambda r, c: (0, 0, 0)),
            pl.BlockSpec((1, cin, F, F), lambda r, c: (c, 0, 0, 0)),
        ],
        out_specs=pl.BlockSpec((1, RB, F), lambda r, c: (c, r, 0)),
        out_shape=jax.ShapeDtypeStruct((cout, N, F), jnp.float32),
    )(xc, dinv, b_in, w_chunks)


def _final_body(xc_ref, dinv_ref, b_ref, out_ref):
    dinv = dinv_ref[...]
    out_ref[:, 0:F] = xc_ref[0] * dinv + b_ref[0]
    out_ref[:, F:2 * F] = xc_ref[1] * dinv + b_ref[1]


def _final(xc, dinv, b_out):
    return pl.pallas_call(
        _final_body,
        grid=(N // RB,),
        in_specs=[
            pl.BlockSpec((2, RB, F), lambda r: (0, r, 0)),
            pl.BlockSpec((RB, 1), lambda r: (r, 0)),
            pl.BlockSpec((2, 1, F), lambda r: (0, 0, 0)),
        ],
        out_specs=pl.BlockSpec((RB, 2 * F), lambda r: (r, 0)),
        out_shape=jax.ShapeDtypeStruct((N, 2 * F), jnp.float32),
    )(xc, dinv, b_out)


def _pad_edges(arr, per, pad_to, pad_vals):
    g = arr.shape[0]
    return jnp.concatenate(
        [arr, jnp.broadcast_to(pad_vals, (g, pad_to - per))], axis=1)


def kernel(x, edge_index, W1, b1, W2, b2, W3, b3):
    src = edge_index[0].astype(jnp.int32)
    dst = edge_index[1].astype(jnp.int32)

    npadp = EPT_PAD - EPT
    srcp_pad = (jnp.arange(npadp, dtype=jnp.int32) * 37) % N
    dstp_pad = N + (jnp.arange(npadp, dtype=jnp.int32) % 16)
    src_t = _pad_edges(src.reshape(NS, EPT), EPT, EPT_PAD, srcp_pad)
    dst_t = _pad_edges(dst.reshape(NS, EPT), EPT, EPT_PAD, dstp_pad)
    dst_rs = dst_t.reshape(NS, NBLK, WPB, W_PROP)
    src_rs = src_t.reshape(NS, NBLK, WPB, W_PROP)
    src4 = (src_rs[None] + (jnp.arange(4, dtype=jnp.int32)
                            * N)[:, None, None, None, None])
    src2 = src4[:2]

    npadd = EPW_PAD - EPW
    dstd_pad = N + (jnp.arange(npadd, dtype=jnp.int32) % 16)
    dst_deg = _pad_edges(dst.reshape(NC * NS, EPW), EPW, EPW_PAD,
                         dstd_pad).reshape(NC * NS, NW_DEG, W_DEG)

    ones_wide = jnp.zeros((W_DEG, F), jnp.float32).at[:, 0].set(1.0)
    zeros_wide = jnp.zeros((N, F), jnp.float32)

    degp = _deg_call(dst_deg, ones_wide, zeros_wide)
    w1c = W1.reshape(256, 4, F).transpose(1, 0, 2)
    h1u = _mm1(x, w1c)
    h1, dinv = _scale1(h1u, degp)
    a1 = _prop4(h1.reshape(4 * N, F), src4, dst_rs)

    w2c = W2.reshape(4, F, 4, F).transpose(2, 0, 1, 3)
    h2 = _mm23(a1, dinv, b1.reshape(4, 1, F), w2c)
    a2 = _prop4(h2.reshape(4 * N, F), src4, dst_rs)

    w3c = W3.reshape(4, F, 2, F).transpose(2, 0, 1, 3)
    h3 = _mm23(a2, dinv, b2.reshape(4, 1, F), w3c)
    a3 = _prop2(h3.reshape(2 * N, F), src2, dst_rs)

    return _final(a3, dinv, b3.reshape(2, 1, F))


# speedup vs baseline: 1.0741x; 1.0741x over previous
import jax
import jax.numpy as jnp
from jax import lax
from jax.experimental import pallas as pl
from jax.experimental.pallas import tpu as pltpu
from jax.experimental.pallas import tpu_sc as plsc

N = 10000
E = 160000
NS = 16
NC = 2
RB = 1000
F = 128

NJ = N + 16
RPT = 624
TAIL = N - NS * RPT
TAIL_OFF = NS * RPT

EPT = E // NS
W_PROP = 80
NB = 4
WPB = 16
NBLK = 8
EPT_PAD = W_PROP * WPB * NBLK
NW_PROP = WPB * NBLK

W_DEG = 128
EPW = E // (NC * NS)
EPW_PAD = 5120
NW_DEG = EPW_PAD // W_DEG

_mesh = plsc.VectorSubcoreMesh(core_axis_name="c", subcore_axis_name="s")


def _copy_rows(sid, mk_src, mk_dst):
    pltpu.sync_copy(mk_src(sid * RPT, RPT), mk_dst(sid * RPT, RPT))

    @pl.when(sid == 0)
    def _():
        pltpu.sync_copy(mk_src(TAIL_OFF, TAIL), mk_dst(TAIL_OFF, TAIL))


def _deg_body(dst_hbm, ones_hbm, zeros_hbm, out_hbm, dstv, onesv, deg_s, sem):
    cid = lax.axis_index("c")
    sid = lax.axis_index("s")
    wid = cid * NS + sid
    pltpu.sync_copy(ones_hbm, onesv)
    pltpu.sync_copy(dst_hbm.at[wid], dstv)
    _copy_rows(sid, lambda o, n: zeros_hbm.at[pl.ds(o, n)],
               lambda o, n: deg_s.at[pl.ds(o, n)])
    plsc.subcore_barrier()

    descs = []
    for w in range(NW_DEG):
        descs.append(pltpu.async_copy(onesv, deg_s.at[dstv.at[w]], sem,
                                      add=True))
    for d in descs:
        d.wait()
    plsc.subcore_barrier()
    _copy_rows(sid, lambda o, n: deg_s.at[pl.ds(o, n)],
               lambda o, n: out_hbm.at[cid, pl.ds(o, n)])


_deg_call = pl.kernel(
    _deg_body,
    out_type=jax.ShapeDtypeStruct((NC, N, F), jnp.float32),
    mesh=_mesh,
    scratch_types=[
        pltpu.VMEM((NW_DEG, W_DEG), jnp.int32),
        pltpu.VMEM((W_DEG, F), jnp.float32),
        pltpu.VMEM_SHARED((NJ, F), jnp.float32),
        pltpu.SemaphoreType.DMA,
    ],
)


def _make_prop(C):
    CPC = C // NC

    def body(h_hbm, srcs_hbm, dst_hbm, out_hbm,
             s0s, s0d, s1s, s1d, r0, r1, r2, r3, accum_s,
             gs0, gs1, gs2, gs3, ss0, ss1, ss2, ss3, is0, is1):
        cid = lax.axis_index("c")
        sid = lax.axis_index("s")
        bufs = ((r0, gs0, ss0), (r1, gs1, ss1), (r2, gs2, ss2),
                (r3, gs3, ss3))

        for k in range(CPC):
            chunk = cid * CPC + k

            def ld_blk(b, ssrc, sdst, sem):
                return (pltpu.make_async_copy(srcs_hbm.at[chunk, sid, b],
                                              ssrc, sem),
                        pltpu.make_async_copy(dst_hbm.at[sid, b], sdst, sem))

            def triple(ssrc, sdst, r, nxt):
                for j, (buf, gsem, ssem) in enumerate(bufs):
                    pltpu.make_async_copy(h_hbm.at[ssrc.at[r + j]], buf,
                                          gsem).wait()
                    pltpu.async_copy(buf, accum_s.at[sdst.at[r + j]], ssem,
                                     add=True)
                for j, (buf, gsem, ssem) in enumerate(bufs):
                    pltpu.make_async_copy(buf, accum_s.at[sdst.at[r + j]],
                                          ssem).wait()
                    if nxt is not None:
                        nsrc, nr = nxt
                        pltpu.async_copy(h_hbm.at[nsrc.at[nr + j]], buf,
                                         gsem)

            def block(ssrc, sdst, nxt_wait, cross):
                for t in range(WPB // NB - 1):
                    triple(ssrc, sdst, NB * t, (ssrc, NB * t + NB))
                if nxt_wait is not None:
                    for d in nxt_wait:
                        d.wait()
                triple(ssrc, sdst, WPB - NB, cross)

            _copy_rows(sid, lambda o, n: h_hbm.at[pl.ds(chunk * N + o, n)],
                       lambda o, n: accum_s.at[pl.ds(o, n)])
            plsc.subcore_barrier()

            for d in ld_blk(0, s0s, s0d, is0):
                d.start()
            for d in ld_blk(0, s0s, s0d, is0):
                d.wait()
            for d in ld_blk(1, s1s, s1d, is1):
                d.start()
            for j, (buf, gsem, _) in enumerate(bufs):
                pltpu.async_copy(h_hbm.at[s0s.at[j]], buf, gsem)

            block(s0s, s0d, ld_blk(1, s1s, s1d, is1), (s1s, 0))

            def fbody(i, carry):
                b1 = 2 * i + 1
                for d in ld_blk(b1 + 1, s0s, s0d, is0):
                    d.start()
                block(s1s, s1d, ld_blk(b1 + 1, s0s, s0d, is0), (s0s, 0))
                b2 = 2 * i + 2
                for d in ld_blk(b2 + 1, s1s, s1d, is1):
                    d.start()
                block(s0s, s0d, ld_blk(b2 + 1, s1s, s1d, is1), (s1s, 0))
                return carry

            lax.fori_loop(0, (NBLK - 2) // 2, fbody, 0)

            block(s1s, s1d, None, None)

            plsc.subcore_barrier()
            _copy_rows(sid, lambda o, n: accum_s.at[pl.ds(o, n)],
                       lambda o, n: out_hbm.at[chunk, pl.ds(o, n)])
            plsc.subcore_barrier()

    return pl.kernel(
        body,
        out_type=jax.ShapeDtypeStruct((C, N, F), jnp.float32),
        mesh=_mesh,
        scratch_types=[
            pltpu.VMEM((WPB, W_PROP), jnp.int32),
            pltpu.VMEM((WPB, W_PROP), jnp.int32),
            pltpu.VMEM((WPB, W_PROP), jnp.int32),
            pltpu.VMEM((WPB, W_PROP), jnp.int32),
            pltpu.VMEM((W_PROP, F), jnp.float32),
            pltpu.VMEM((W_PROP, F), jnp.float32),
            pltpu.VMEM((W_PROP, F), jnp.float32),
            pltpu.VMEM((W_PROP, F), jnp.float32),
            pltpu.VMEM_SHARED((NJ, F), jnp.float32),
            pltpu.SemaphoreType.DMA,
            pltpu.SemaphoreType.DMA,
            pltpu.SemaphoreType.DMA,
            pltpu.SemaphoreType.DMA,
            pltpu.SemaphoreType.DMA,
            pltpu.SemaphoreType.DMA,
            pltpu.SemaphoreType.DMA,
            pltpu.SemaphoreType.DMA,
            pltpu.SemaphoreType.DMA,
            pltpu.SemaphoreType.DMA,
        ],
    )


_prop4 = _make_prop(4)
_prop2 = _make_prop(2)


def _mm1_body(x_ref, w_ref, out_ref):
    out_ref[0] = jnp.dot(x_ref[...], w_ref[0],
                         preferred_element_type=jnp.float32)


def _mm1(x, w_chunks):
    cout = w_chunks.shape[0]
    return pl.pallas_call(
        _mm1_body,
        grid=(N // RB, cout),
        in_specs=[
            pl.BlockSpec((RB, 256), lambda r, c: (r, 0)),
            pl.BlockSpec((1, 256, F), lambda r, c: (c, 0, 0)),
        ],
        out_specs=pl.BlockSpec((1, RB, F), lambda r, c: (c, r, 0)),
        out_shape=jax.ShapeDtypeStruct((cout, N, F), jnp.float32),
    )(x, w_chunks)


def _scale1_body(h_ref, degp_ref, hout_ref, dinv_ref):
    d = 1.0 + degp_ref[0, :, 0:1] + degp_ref[1, :, 0:1]
    dinv = lax.rsqrt(d)
    hout_ref[0] = h_ref[0] * dinv
    dinv_ref[...] = dinv


def _scale1(h, degp):
    cout = h.shape[0]
    return pl.pallas_call(
        _scale1_body,
        grid=(N // RB, cout),
        in_specs=[
            pl.BlockSpec((1, RB, F), lambda r, c: (c, r, 0)),
            pl.BlockSpec((NC, RB, F), lambda r, c: (0, r, 0)),
        ],
        out_specs=[
            pl.BlockSpec((1, RB, F), lambda r, c: (c, r, 0)),
            pl.BlockSpec((RB, 1), lambda r, c: (r, 0)),
        ],
        out_shape=[
            jax.ShapeDtypeStruct((cout, N, F), jnp.float32),
            jax.ShapeDtypeStruct((N, 1), jnp.float32),
        ],
    )(h, degp)


def _make_mm23_body(cin):
    def body(xc_ref, dinv_ref, b_ref, w_ref, out_ref):
        dinv = dinv_ref[...]
        acc = jnp.zeros((RB, F), jnp.float32)
        for k in range(cin):
            xk = jnp.maximum(xc_ref[k] * dinv + b_ref[k], 0.0)
            acc = acc + jnp.dot(xk, w_ref[0, k],
                                preferred_element_type=jnp.float32)
        out_ref[0] = acc * dinv
    return body


def _mm23(xc, dinv, b_in, w_chunks):
    cout, cin = w_chunks.shape[0], w_chunks.shape[1]
    return pl.pallas_call(
        _make_mm23_body(cin),
        grid=(N // RB, cout),
        in_specs=[
            pl.BlockSpec((cin, RB, F), lambda r, c: (0, r, 0)),
            pl.BlockSpec((RB, 1), lambda r, c: (r, 0)),
            pl.BlockSpec((cin, 1, F), lambda r, c: (0, 0, 0)),
            pl.BlockSpec((1, cin, F, F), lambda r, c: (c, 0, 0, 0)),
        ],
        out_specs=pl.BlockSpec((1, RB, F), lambda r, c: (c, r, 0)),
        out_shape=jax.ShapeDtypeStruct((cout, N, F), jnp.float32),
    )(xc, dinv, b_in, w_chunks)


def _final_body(xc_ref, dinv_ref, b_ref, out_ref):
    dinv = dinv_ref[...]
    out_ref[:, 0:F] = xc_ref[0] * dinv + b_ref[0]
    out_ref[:, F:2 * F] = xc_ref[1] * dinv + b_ref[1]


def _final(xc, dinv, b_out):
    return pl.pallas_call(
        _final_body,
        grid=(N // RB,),
        in_specs=[
            pl.BlockSpec((2, RB, F), lambda r: (0, r, 0)),
            pl.BlockSpec((RB, 1), lambda r: (r, 0)),
            pl.BlockSpec((2, 1, F), lambda r: (0, 0, 0)),
        ],
        out_specs=pl.BlockSpec((RB, 2 * F), lambda r: (r, 0)),
        out_shape=jax.ShapeDtypeStruct((N, 2 * F), jnp.float32),
    )(xc, dinv, b_out)


def _pad_edges(arr, per, pad_to, pad_vals):
    g = arr.shape[0]
    return jnp.concatenate(
        [arr, jnp.broadcast_to(pad_vals, (g, pad_to - per))], axis=1)


def kernel(x, edge_index, W1, b1, W2, b2, W3, b3):
    src = edge_index[0].astype(jnp.int32)
    dst = edge_index[1].astype(jnp.int32)

    npadp = EPT_PAD - EPT
    srcp_pad = (jnp.arange(npadp, dtype=jnp.int32) * 37) % N
    dstp_pad = N + (jnp.arange(npadp, dtype=jnp.int32) % 16)
    src_t = _pad_edges(src.reshape(NS, EPT), EPT, EPT_PAD, srcp_pad)
    dst_t = _pad_edges(dst.reshape(NS, EPT), EPT, EPT_PAD, dstp_pad)
    dst_rs = dst_t.reshape(NS, NBLK, WPB, W_PROP)
    src_rs = src_t.reshape(NS, NBLK, WPB, W_PROP)
    src4 = (src_rs[None] + (jnp.arange(4, dtype=jnp.int32)
                            * N)[:, None, None, None, None])
    src2 = src4[:2]

    npadd = EPW_PAD - EPW
    dstd_pad = N + (jnp.arange(npadd, dtype=jnp.int32) % 16)
    dst_deg = _pad_edges(dst.reshape(NC * NS, EPW), EPW, EPW_PAD,
                         dstd_pad).reshape(NC * NS, NW_DEG, W_DEG)

    ones_wide = jnp.zeros((W_DEG, F), jnp.float32).at[:, 0].set(1.0)
    zeros_wide = jnp.zeros((N, F), jnp.float32)

    degp = _deg_call(dst_deg, ones_wide, zeros_wide)
    w1c = W1.reshape(256, 4, F).transpose(1, 0, 2)
    h1u = _mm1(x, w1c)
    h1, dinv = _scale1(h1u, degp)
    a1 = _prop4(h1.reshape(4 * N, F), src4, dst_rs)

    w2c = W2.reshape(4, F, 4, F).transpose(2, 0, 1, 3)
    h2 = _mm23(a1, dinv, b1.reshape(4, 1, F), w2c)
    a2 = _prop4(h2.reshape(4 * N, F), src4, dst_rs)

    w3c = W3.reshape(4, F, 2, F).transpose(2, 0, 1, 3)
    h3 = _mm23(a2, dinv, b2.reshape(4, 1, F), w3c)
    a3 = _prop2(h3.reshape(2 * N, F), src2, dst_rs)

    return _final(a3, dinv, b3.reshape(2, 1, F))
